# baseline (speedup 1.0000x reference)
import jax
import jax.numpy as jnp
from jax import lax
from jax.experimental import pallas as pl
from jax.experimental.pallas import tpu as pltpu
from jax.experimental.pallas import tpu_sc as plsc

_N_TABLES = 26
_VOCAB = 100000
_WIDTH = 32
_BATCH = 4096

_NC = 2
_NS = 16
_L = 16
_NW = _NC * _NS
_BPW = _BATCH // _NW


def _body(tab_hbm, x_hbm, out_hbm, idx_v, tidx_v, rows_v, sem):
    wid = lax.axis_index("s") * _NC + lax.axis_index("c")
    base = wid * _BPW

    pltpu.sync_copy(x_hbm.at[pl.ds(base, _BPW)], idx_v)

    def compute_tidx(t, carry):
        off = t * _VOCAB
        for j in range(_BPW // _L):
            sl = pl.ds(j * _L, _L)
            tidx_v[t, sl] = idx_v[sl] + off
        return carry

    lax.fori_loop(0, _N_TABLES, compute_tidx, 0)

    def fire(t, carry):
        pltpu.async_copy(tab_hbm.at[tidx_v.at[t]], rows_v.at[t], sem)
        return carry

    lax.fori_loop(0, _N_TABLES, fire, 0)

    def drain(t, carry):
        pltpu.make_async_copy(tab_hbm.at[tidx_v.at[t]], rows_v.at[t], sem).wait()
        return carry

    lax.fori_loop(0, _N_TABLES, drain, 0)

    pltpu.sync_copy(rows_v, out_hbm.at[:, pl.ds(base, _BPW), :])


@jax.jit
def _lookup(tables_flat, x):
    mesh = plsc.VectorSubcoreMesh(core_axis_name="c", subcore_axis_name="s")
    return pl.kernel(
        _body,
        out_type=jax.ShapeDtypeStruct((_N_TABLES, _BATCH, _WIDTH), jnp.float32),
        mesh=mesh,
        scratch_types=[
            pltpu.VMEM((_BPW,), jnp.int32),
            pltpu.VMEM((_N_TABLES, _BPW), jnp.int32),
            pltpu.VMEM((_N_TABLES, _BPW, _WIDTH), jnp.float32),
            pltpu.SemaphoreType.DMA,
        ],
        compiler_params=pltpu.CompilerParams(use_tc_tiling_on_sc=False),
    )(tables_flat, x)


def kernel(x, tables):
    tables_flat = tables.reshape(_N_TABLES * _VOCAB, _WIDTH)
    return _lookup(tables_flat, x.astype(jnp.int32))
